# Initial kernel scaffold; baseline (speedup 1.0000x reference)
import jax
import jax.numpy as jnp
from jax.experimental import pallas as pl


def kernel(inputs, tables, w_wide, b_wide, deep_Ws, deep_Bs):
    raise NotImplementedError("write your pallas kernel here")


# speedup vs baseline: 1.1462x; 1.1462x over previous
import functools

import jax
import jax.numpy as jnp
from jax import lax
from jax.experimental import pallas as pl
from jax.experimental.pallas import tpu as pltpu
from jax.experimental.pallas import tpu_sc as plsc

_BATCH = 4096
_N_DENSE = 13
_N_SPARSE = 26
_VOCAB = 100000
_EMBED = 32

_NC = 2
_NS = 16
_NW = _NC * _NS
_ROWS = _BATCH * _N_SPARSE
_ROWS_PER_W = _ROWS // _NW
_CH = 128
_NCH = _ROWS_PER_W // _CH


def _sc_gather(table_flat, idx3):
    mesh = plsc.VectorSubcoreMesh(core_axis_name="c", subcore_axis_name="s")

    @functools.partial(
        pl.kernel,
        mesh=mesh,
        out_type=jax.ShapeDtypeStruct((_NW, _NCH, _CH, _EMBED), jnp.float32),
        scratch_types=[
            pltpu.VMEM((_NCH, _CH), jnp.int32),
            pltpu.VMEM((_NCH, _CH, _EMBED), jnp.float32),
            pltpu.SemaphoreType.DMA,
        ],
        compiler_params=pltpu.CompilerParams(use_tc_tiling_on_sc=False),
    )
    def k(table_hbm, idx_hbm, out_hbm, idx_v, rows_v, sem):
        wid = lax.axis_index("s") * _NC + lax.axis_index("c")
        pltpu.sync_copy(idx_hbm.at[wid], idx_v)
        copies = [
            pltpu.async_copy(table_hbm.at[idx_v.at[j]], rows_v.at[j], sem)
            for j in range(_NCH)
        ]
        for c in copies:
            c.wait()
        pltpu.sync_copy(rows_v, out_hbm.at[wid])

    return k(table_flat, idx3)


def _mlp_body(x_ref, inp_ref, w1_ref, b1_ref, w2_ref, b2_ref, w3_ref, b3_ref,
              w4_ref, wfull_ref, c0_ref, out_ref):
    f32 = jnp.float32
    h = lax.dot_general(x_ref[...], w1_ref[...], (((1,), (0,)), ((), ())),
                        preferred_element_type=f32)
    h = jnp.maximum(h + b1_ref[...], 0.0)
    h = lax.dot_general(h, w2_ref[...], (((1,), (0,)), ((), ())),
                        preferred_element_type=f32)
    h = jnp.maximum(h + b2_ref[...], 0.0)
    h = lax.dot_general(h, w3_ref[...], (((1,), (0,)), ((), ())),
                        preferred_element_type=f32)
    h = jnp.maximum(h + b3_ref[...], 0.0)
    deep = lax.dot_general(h, w4_ref[...], (((1,), (0,)), ((), ())),
                           preferred_element_type=f32)
    wide = lax.dot_general(inp_ref[...], wfull_ref[...], (((1,), (0,)), ((), ())),
                           preferred_element_type=f32)
    z = 0.5 * (deep + wide + c0_ref[0, 0])
    out_ref[...] = 1.0 / (1.0 + jnp.exp(-z))


def _tc_mlp(emb, inputs, w1, b1, w2, b2, w3, b3, w4, wfull, c0):
    bb = 512
    nb = _BATCH // bb
    d_in = inputs.shape[1]
    k1 = emb.shape[1]
    h1, h2, h3 = w1.shape[1], w2.shape[1], w3.shape[1]
    return pl.pallas_call(
        _mlp_body,
        grid=(nb,),
        in_specs=[
            pl.BlockSpec((bb, k1), lambda i: (i, 0)),
            pl.BlockSpec((bb, d_in), lambda i: (i, 0)),
            pl.BlockSpec((k1, h1), lambda i: (0, 0)),
            pl.BlockSpec((1, h1), lambda i: (0, 0)),
            pl.BlockSpec((h1, h2), lambda i: (0, 0)),
            pl.BlockSpec((1, h2), lambda i: (0, 0)),
            pl.BlockSpec((h2, h3), lambda i: (0, 0)),
            pl.BlockSpec((1, h3), lambda i: (0, 0)),
            pl.BlockSpec((h3, 1), lambda i: (0, 0)),
            pl.BlockSpec((d_in, 1), lambda i: (0, 0)),
            pl.BlockSpec((1, 1), lambda i: (0, 0)),
        ],
        out_specs=pl.BlockSpec((bb, 1), lambda i: (i, 0)),
        out_shape=jax.ShapeDtypeStruct((_BATCH, 1), jnp.float32),
    )(emb, inputs, w1, b1, w2, b2, w3, b3, w4, wfull, c0)


def kernel(inputs, tables, w_wide, b_wide, deep_Ws, deep_Bs):
    sparse_idx = inputs[:, _N_DENSE:_N_DENSE + _N_SPARSE].astype(jnp.int32)
    offs = (jnp.arange(_N_SPARSE, dtype=jnp.int32) * _VOCAB)[None, :]
    flat_idx = (sparse_idx + offs).reshape(_NW, _NCH, _CH)
    table_flat = tables.reshape(_N_SPARSE * _VOCAB, _EMBED)

    wfull = jnp.concatenate(
        [w_wide[:_N_DENSE],
         jnp.zeros((_N_SPARSE, 1), jnp.float32),
         w_wide[_N_DENSE:]], axis=0)
    w4 = deep_Ws[3]
    c0 = (b_wide[0] + deep_Bs[3][0]).reshape(1, 1)
    b1 = deep_Bs[0].reshape(1, -1)
    b2 = deep_Bs[1].reshape(1, -1)
    b3 = deep_Bs[2].reshape(1, -1)

    emb = _sc_gather(table_flat, flat_idx).reshape(_BATCH, _N_SPARSE * _EMBED)

    return _tc_mlp(emb, inputs, deep_Ws[0], b1, deep_Ws[1], b2,
                   deep_Ws[2], b3, w4, wfull, c0)
